# Initial kernel scaffold; baseline (speedup 1.0000x reference)
import jax
import jax.numpy as jnp
from jax.experimental import pallas as pl


def kernel(hidden_states, Wg, bg, W_gate_up, b_gate_up, W_down, b_down):
    raise NotImplementedError("write your pallas kernel here")


# speedup vs baseline: 1.8858x; 1.8858x over previous
import jax
import jax.numpy as jnp
from jax.experimental import pallas as pl
from jax.experimental.pallas import tpu as pltpu

NUM_TOKENS = 2048
D_MODEL = 1024
D_FF = 2048
NUM_EXPERTS = 8
TOP_K = 2
ALPHA = 1.702
BETA = 1.0

BF = 256


def _moe_kernel(x_ref, wg_ref, bg_ref, wgate_ref, bgate_ref, wup_ref, bup_ref,
                wdown_ref, bdown_ref, out_ref):
    e = pl.program_id(0)
    f = pl.program_id(1)
    x = x_ref[...]

    logits = jnp.dot(x, wg_ref[...], preferred_element_type=jnp.float32) + bg_ref[...]
    p = jax.nn.softmax(logits, axis=-1)
    a1 = jnp.argmax(p, axis=-1)
    p1 = jnp.max(p, axis=-1)
    ids = jax.lax.broadcasted_iota(jnp.int32, p.shape, 1)
    masked = jnp.where(ids == a1[:, None], -jnp.inf, p)
    a2 = jnp.argmax(masked, axis=-1)
    p2 = jnp.max(masked, axis=-1)
    denom = p1 + p2
    coef = jnp.where(a1 == e, p1, jnp.where(a2 == e, p2, 0.0)) / denom

    gate = jnp.dot(x, wgate_ref[0], preferred_element_type=jnp.float32) + bgate_ref[0]
    up = jnp.dot(x, wup_ref[0], preferred_element_type=jnp.float32) + bup_ref[0]
    act = (up + BETA) * (gate * jax.nn.sigmoid(ALPHA * gate))
    partial = jnp.dot(act, wdown_ref[0], preferred_element_type=jnp.float32)

    contrib = coef[:, None] * (partial + jnp.where(f == 0, 1.0, 0.0) * bdown_ref[0])

    @pl.when((e == 0) & (f == 0))
    def _():
        out_ref[...] = contrib

    @pl.when((e != 0) | (f != 0))
    def _():
        out_ref[...] += contrib


@jax.jit
def kernel(hidden_states, Wg, bg, W_gate_up, b_gate_up, W_down, b_down):
    W_gate = W_gate_up[:, :, 0::2]
    W_up = W_gate_up[:, :, 1::2]
    b_gate = b_gate_up[:, 0::2].reshape(NUM_EXPERTS, 1, D_FF)
    b_up = b_gate_up[:, 1::2].reshape(NUM_EXPERTS, 1, D_FF)
    b_down2 = b_down.reshape(NUM_EXPERTS, 1, D_MODEL)
    bg2 = bg.reshape(1, NUM_EXPERTS)

    nf = D_FF // BF
    grid = (NUM_EXPERTS, nf)
    out = pl.pallas_call(
        _moe_kernel,
        grid=grid,
        in_specs=[
            pl.BlockSpec((NUM_TOKENS, D_MODEL), lambda e, f: (0, 0)),
            pl.BlockSpec((D_MODEL, NUM_EXPERTS), lambda e, f: (0, 0)),
            pl.BlockSpec((1, NUM_EXPERTS), lambda e, f: (0, 0)),
            pl.BlockSpec((1, D_MODEL, BF), lambda e, f: (e, 0, f)),
            pl.BlockSpec((1, 1, BF), lambda e, f: (e, 0, f)),
            pl.BlockSpec((1, D_MODEL, BF), lambda e, f: (e, 0, f)),
            pl.BlockSpec((1, 1, BF), lambda e, f: (e, 0, f)),
            pl.BlockSpec((1, BF, D_MODEL), lambda e, f: (e, f, 0)),
            pl.BlockSpec((1, 1, D_MODEL), lambda e, f: (e, 0, 0)),
        ],
        out_specs=pl.BlockSpec((NUM_TOKENS, D_MODEL), lambda e, f: (0, 0)),
        out_shape=jax.ShapeDtypeStruct((NUM_TOKENS, D_MODEL), jnp.float32),
    )(hidden_states, Wg, bg2, W_gate, b_gate, W_up, b_up, W_down, b_down2)
    return out


# speedup vs baseline: 3.2441x; 1.7203x over previous
import jax
import jax.numpy as jnp
from jax.experimental import pallas as pl
from jax.experimental.pallas import tpu as pltpu

NUM_TOKENS = 2048
D_MODEL = 1024
D_FF = 2048
NUM_EXPERTS = 8
TOP_K = 2
ALPHA = 1.702
BETA = 1.0

BF = 256


def _moe_kernel(x_ref, wg_ref, bg_ref, wgate_ref, bgate_ref, wup_ref, bup_ref,
                wdown_ref, bdown_ref, out_ref):
    e = pl.program_id(0)
    f = pl.program_id(1)
    x = x_ref[...]

    logits = jnp.dot(x, wg_ref[...], preferred_element_type=jnp.float32) + bg_ref[...]
    p = jax.nn.softmax(logits, axis=-1)
    a1 = jnp.argmax(p, axis=-1)
    p1 = jnp.max(p, axis=-1)
    ids = jax.lax.broadcasted_iota(jnp.int32, p.shape, 1)
    masked = jnp.where(ids == a1[:, None], -jnp.inf, p)
    a2 = jnp.argmax(masked, axis=-1)
    p2 = jnp.max(masked, axis=-1)
    denom = p1 + p2
    coef = jnp.where(a1 == e, p1, jnp.where(a2 == e, p2, 0.0)) / denom

    xb = x.astype(jnp.bfloat16)
    gate = jnp.dot(xb, wgate_ref[0], preferred_element_type=jnp.float32) + bgate_ref[0]
    up = jnp.dot(xb, wup_ref[0], preferred_element_type=jnp.float32) + bup_ref[0]
    act = (up + BETA) * (gate * jax.nn.sigmoid(ALPHA * gate))
    partial = jnp.dot(act.astype(jnp.bfloat16), wdown_ref[0],
                      preferred_element_type=jnp.float32)

    contrib = coef[:, None] * (partial + jnp.where(f == 0, 1.0, 0.0) * bdown_ref[0])

    @pl.when((e == 0) & (f == 0))
    def _():
        out_ref[...] = contrib

    @pl.when((e != 0) | (f != 0))
    def _():
        out_ref[...] += contrib


@jax.jit
def kernel(hidden_states, Wg, bg, W_gate_up, b_gate_up, W_down, b_down):
    W_gate = W_gate_up[:, :, 0::2].astype(jnp.bfloat16)
    W_up = W_gate_up[:, :, 1::2].astype(jnp.bfloat16)
    W_down_b = W_down.astype(jnp.bfloat16)
    b_gate = b_gate_up[:, 0::2].reshape(NUM_EXPERTS, 1, D_FF)
    b_up = b_gate_up[:, 1::2].reshape(NUM_EXPERTS, 1, D_FF)
    b_down2 = b_down.reshape(NUM_EXPERTS, 1, D_MODEL)
    bg2 = bg.reshape(1, NUM_EXPERTS)

    nf = D_FF // BF
    grid = (NUM_EXPERTS, nf)
    out = pl.pallas_call(
        _moe_kernel,
        grid=grid,
        in_specs=[
            pl.BlockSpec((NUM_TOKENS, D_MODEL), lambda e, f: (0, 0)),
            pl.BlockSpec((D_MODEL, NUM_EXPERTS), lambda e, f: (0, 0)),
            pl.BlockSpec((1, NUM_EXPERTS), lambda e, f: (0, 0)),
            pl.BlockSpec((1, D_MODEL, BF), lambda e, f: (e, 0, f)),
            pl.BlockSpec((1, 1, BF), lambda e, f: (e, 0, f)),
            pl.BlockSpec((1, D_MODEL, BF), lambda e, f: (e, 0, f)),
            pl.BlockSpec((1, 1, BF), lambda e, f: (e, 0, f)),
            pl.BlockSpec((1, BF, D_MODEL), lambda e, f: (e, f, 0)),
            pl.BlockSpec((1, 1, D_MODEL), lambda e, f: (e, 0, 0)),
        ],
        out_specs=pl.BlockSpec((NUM_TOKENS, D_MODEL), lambda e, f: (0, 0)),
        out_shape=jax.ShapeDtypeStruct((NUM_TOKENS, D_MODEL), jnp.float32),
    )(hidden_states, Wg, bg2, W_gate, b_gate, W_up, b_up, W_down_b, b_down2)
    return out


# speedup vs baseline: 15.6321x; 4.8187x over previous
import jax
import jax.numpy as jnp
from jax.experimental import pallas as pl
from jax.experimental.pallas import tpu as pltpu

NUM_TOKENS = 2048
D_MODEL = 1024
D_FF = 2048
NUM_EXPERTS = 8
TOP_K = 2
ALPHA = 1.702
BETA = 1.0

BF = 256


def _moe_kernel(x_ref, wg_ref, bg_ref, wf_ref, bf_ref, wdown_ref, bdown_ref,
                out_ref):
    e = pl.program_id(0)
    f = pl.program_id(1)
    x = x_ref[...]

    logits = jnp.dot(x, wg_ref[...], preferred_element_type=jnp.float32) + bg_ref[...]
    p = jax.nn.softmax(logits, axis=-1)
    a1 = jnp.argmax(p, axis=-1)
    p1 = jnp.max(p, axis=-1)
    ids = jax.lax.broadcasted_iota(jnp.int32, p.shape, 1)
    masked = jnp.where(ids == a1[:, None], -jnp.inf, p)
    a2 = jnp.argmax(masked, axis=-1)
    p2 = jnp.max(masked, axis=-1)
    denom = p1 + p2
    coef = jnp.where(a1 == e, p1, jnp.where(a2 == e, p2, 0.0)) / denom

    i0 = jax.lax.broadcasted_iota(jnp.int32, (2 * BF, BF), 0)
    i1 = jax.lax.broadcasted_iota(jnp.int32, (2 * BF, BF), 1)
    se = (i0 == 2 * i1).astype(jnp.bfloat16)
    so = (i0 == 2 * i1 + 1).astype(jnp.bfloat16)

    wf = wf_ref[0]
    w_gate = jnp.dot(wf, se, preferred_element_type=jnp.float32).astype(jnp.bfloat16)
    w_up = jnp.dot(wf, so, preferred_element_type=jnp.float32).astype(jnp.bfloat16)
    bf_row = bf_ref[0]
    b_gate = jnp.dot(bf_row, se.astype(jnp.float32),
                     preferred_element_type=jnp.float32)
    b_up = jnp.dot(bf_row, so.astype(jnp.float32),
                   preferred_element_type=jnp.float32)

    xb = x.astype(jnp.bfloat16)
    gate = jnp.dot(xb, w_gate, preferred_element_type=jnp.float32) + b_gate
    up = jnp.dot(xb, w_up, preferred_element_type=jnp.float32) + b_up
    act = (up + BETA) * (gate * jax.nn.sigmoid(ALPHA * gate))
    partial = jnp.dot(act.astype(jnp.bfloat16), wdown_ref[0],
                      preferred_element_type=jnp.float32)

    contrib = coef[:, None] * (partial + jnp.where(f == 0, 1.0, 0.0) * bdown_ref[0])

    @pl.when((e == 0) & (f == 0))
    def _():
        out_ref[...] = contrib

    @pl.when((e != 0) | (f != 0))
    def _():
        out_ref[...] += contrib


@jax.jit
def kernel(hidden_states, Wg, bg, W_gate_up, b_gate_up, W_down, b_down):
    Wf = W_gate_up.astype(jnp.bfloat16)
    bf2 = b_gate_up.reshape(NUM_EXPERTS, 1, 2 * D_FF)
    W_down_b = W_down.astype(jnp.bfloat16)
    b_down2 = b_down.reshape(NUM_EXPERTS, 1, D_MODEL)
    bg2 = bg.reshape(1, NUM_EXPERTS)

    nf = D_FF // BF
    grid = (NUM_EXPERTS, nf)
    out = pl.pallas_call(
        _moe_kernel,
        grid=grid,
        in_specs=[
            pl.BlockSpec((NUM_TOKENS, D_MODEL), lambda e, f: (0, 0)),
            pl.BlockSpec((D_MODEL, NUM_EXPERTS), lambda e, f: (0, 0)),
            pl.BlockSpec((1, NUM_EXPERTS), lambda e, f: (0, 0)),
            pl.BlockSpec((1, D_MODEL, 2 * BF), lambda e, f: (e, 0, f)),
            pl.BlockSpec((1, 1, 2 * BF), lambda e, f: (e, 0, f)),
            pl.BlockSpec((1, BF, D_MODEL), lambda e, f: (e, f, 0)),
            pl.BlockSpec((1, 1, D_MODEL), lambda e, f: (e, 0, 0)),
        ],
        out_specs=pl.BlockSpec((NUM_TOKENS, D_MODEL), lambda e, f: (0, 0)),
        out_shape=jax.ShapeDtypeStruct((NUM_TOKENS, D_MODEL), jnp.float32),
    )(hidden_states, Wg, bg2, Wf, bf2, W_down_b, b_down2)
    return out


# speedup vs baseline: 32.3999x; 2.0726x over previous
import functools

import jax
import jax.numpy as jnp
from jax import lax
from jax.experimental import pallas as pl
from jax.experimental.pallas import tpu as pltpu
from jax.experimental.pallas import tpu_sc as plsc

NUM_TOKENS = 2048
D_MODEL = 1024
D_FF = 2048
NUM_EXPERTS = 8
ALPHA = 1.702
BETA = 1.0

BROW = 128
NBLK = (2 * NUM_TOKENS) // BROW + NUM_EXPERTS
NROWS = NBLK * BROW
NWORK = 32
TPW = NUM_TOKENS // NWORK
DCH = 32


def _cumsum0(x):
    n = x.shape[0]
    k = 1
    while k < n:
        x = x + jnp.concatenate(
            [jnp.zeros((k, x.shape[1]), x.dtype), x[:-k]], axis=0)
        k *= 2
    return x


def _cumsum1(x):
    n = x.shape[1]
    k = 1
    while k < n:
        x = x + jnp.concatenate(
            [jnp.zeros((x.shape[0], k), x.dtype), x[:, :-k]], axis=1)
        k *= 2
    return x

def _router_kernel(x_ref, wg_ref, bg_ref, pos_ref, w_ref, be_ref):
    x = x_ref[...]
    logits = jnp.dot(x, wg_ref[...], preferred_element_type=jnp.float32) + bg_ref[...]
    p = jax.nn.softmax(logits, axis=-1)
    a1 = jnp.argmax(p, axis=-1)
    p1 = jnp.max(p, axis=-1)
    ids = jax.lax.broadcasted_iota(jnp.int32, p.shape, 1)
    masked = jnp.where(ids == a1[:, None], -jnp.inf, p)
    a2 = jnp.argmax(masked, axis=-1)
    p2 = jnp.max(masked, axis=-1)
    denom = p1 + p2
    w1 = p1 / denom
    w2 = p2 / denom

    oh1 = (ids == a1[:, None]).astype(jnp.float32)
    oh2 = (ids == a2[:, None]).astype(jnp.float32)
    both = oh1 + oh2
    s_incl = _cumsum0(both)
    s_excl = s_incl - both
    sizes = s_incl[-1:, :]
    nb = jnp.floor((sizes + (BROW - 1)) / BROW)
    sb = _cumsum1(nb) - nb
    start_row = sb * BROW
    pos1 = jnp.sum(oh1 * (start_row + s_excl), axis=-1, keepdims=True)
    pos2 = jnp.sum(oh2 * (start_row + s_excl), axis=-1, keepdims=True)
    pos_ref[:, 0:1] = pos1.astype(jnp.int32)
    pos_ref[:, 1:2] = pos2.astype(jnp.int32)
    w_ref[:, 0:1] = w1[:, None]
    w_ref[:, 1:2] = w2[:, None]

    cum_end = sb + nb
    bids = jax.lax.broadcasted_iota(
        jnp.int32, (NUM_EXPERTS, NBLK), 1).astype(jnp.float32)
    be = jnp.sum((cum_end.reshape(NUM_EXPERTS, 1) <= bids).astype(jnp.float32),
                 axis=0, keepdims=True)
    be_ref[...] = jnp.minimum(be, NUM_EXPERTS - 1).astype(jnp.int32)


def _router_call(x, Wg, bg2):
    return pl.pallas_call(
        _router_kernel,
        grid=(1,),
        in_specs=[
            pl.BlockSpec((NUM_TOKENS, D_MODEL), lambda i: (0, 0)),
            pl.BlockSpec((D_MODEL, NUM_EXPERTS), lambda i: (0, 0)),
            pl.BlockSpec((1, NUM_EXPERTS), lambda i: (0, 0)),
        ],
        out_specs=[
            pl.BlockSpec((NUM_TOKENS, 2), lambda i: (0, 0)),
            pl.BlockSpec((NUM_TOKENS, 2), lambda i: (0, 0)),
            pl.BlockSpec((1, NBLK), lambda i: (0, 0)),
        ],
        out_shape=[
            jax.ShapeDtypeStruct((NUM_TOKENS, 2), jnp.int32),
            jax.ShapeDtypeStruct((NUM_TOKENS, 2), jnp.float32),
            jax.ShapeDtypeStruct((1, NBLK), jnp.int32),
        ],
    )(x, Wg, bg2)


_sc_mesh = plsc.VectorSubcoreMesh(core_axis_name="c", subcore_axis_name="s")


@functools.partial(
    pl.kernel,
    mesh=_sc_mesh,
    out_type=jax.ShapeDtypeStruct((NROWS, D_MODEL), jnp.float32),
    scratch_types=[
        pltpu.VMEM((2, TPW), jnp.int32),
        pltpu.VMEM((TPW, D_MODEL), jnp.float32),
        pltpu.SemaphoreType.DMA,
    ],
)
def _dispatch_kernel(x_hbm, idx3_hbm, xs_hbm, idx_v, rows_v, sem):
    wid = lax.axis_index("s") * 2 + lax.axis_index("c")
    base = wid * TPW
    pltpu.sync_copy(idx3_hbm.at[wid], idx_v)
    pltpu.sync_copy(x_hbm.at[pl.ds(base, TPW)], rows_v)
    pltpu.async_copy(rows_v, xs_hbm.at[idx_v.at[0]], sem).wait()
    pltpu.async_copy(rows_v, xs_hbm.at[idx_v.at[1]], sem).wait()


def _gm_kernel(be_ref, xs_ref, wf_ref, bf_ref, wd_ref, bd_ref, rows_ref):
    del be_ref
    xb = xs_ref[...].astype(jnp.bfloat16)
    gu = jnp.dot(xb, wf_ref[0], preferred_element_type=jnp.float32) + bf_ref[0]

    i0 = jax.lax.broadcasted_iota(jnp.int32, (512, 256), 0)
    i1 = jax.lax.broadcasted_iota(jnp.int32, (512, 256), 1)
    se = (i0 == 2 * i1).astype(jnp.bfloat16)
    so = (i0 == 2 * i1 + 1).astype(jnp.bfloat16)
    gates = []
    ups = []
    for c in range(2 * D_FF // 512):
        guc = gu[:, c * 512:(c + 1) * 512].astype(jnp.bfloat16)
        gates.append(jnp.dot(guc, se, preferred_element_type=jnp.float32))
        ups.append(jnp.dot(guc, so, preferred_element_type=jnp.float32))
    gate = jnp.concatenate(gates, axis=1)
    up = jnp.concatenate(ups, axis=1)
    act = (up + BETA) * (gate * jax.nn.sigmoid(ALPHA * gate))
    rows_ref[...] = jnp.dot(act.astype(jnp.bfloat16), wd_ref[0],
                            preferred_element_type=jnp.float32) + bd_ref[0]


def _gm_call(be_flat, xs, Wf, bf2, Wd, bd2):
    grid_spec = pltpu.PrefetchScalarGridSpec(
        num_scalar_prefetch=1,
        grid=(NBLK,),
        in_specs=[
            pl.BlockSpec((BROW, D_MODEL), lambda b, be: (b, 0)),
            pl.BlockSpec((1, D_MODEL, 2 * D_FF), lambda b, be: (be[b], 0, 0)),
            pl.BlockSpec((1, 1, 2 * D_FF), lambda b, be: (be[b], 0, 0)),
            pl.BlockSpec((1, D_FF, D_MODEL), lambda b, be: (be[b], 0, 0)),
            pl.BlockSpec((1, 1, D_MODEL), lambda b, be: (be[b], 0, 0)),
        ],
        out_specs=pl.BlockSpec((BROW, D_MODEL), lambda b, be: (b, 0)),
    )
    return pl.pallas_call(
        _gm_kernel,
        grid_spec=grid_spec,
        out_shape=jax.ShapeDtypeStruct((NROWS, D_MODEL), jnp.float32),
    )(be_flat, xs, Wf, bf2, Wd, bd2)


@functools.partial(
    pl.kernel,
    mesh=_sc_mesh,
    out_type=jax.ShapeDtypeStruct((2, NUM_TOKENS, D_MODEL), jnp.float32),
    scratch_types=[
        pltpu.VMEM((2, TPW), jnp.int32),
        pltpu.VMEM((DCH, D_MODEL), jnp.float32),
        pltpu.SemaphoreType.DMA,
    ],
)
def _gather_kernel(rows_hbm, idx3_hbm, g2_hbm, idx_v, r_v, sem):
    wid = lax.axis_index("s") * 2 + lax.axis_index("c")
    base = wid * TPW
    pltpu.sync_copy(idx3_hbm.at[wid], idx_v)
    for k in range(2):
        for c in range(TPW // DCH):
            pltpu.async_copy(
                rows_hbm.at[idx_v.at[k, pl.ds(c * DCH, DCH)]], r_v, sem
            ).wait()
            pltpu.sync_copy(r_v, g2_hbm.at[k, pl.ds(base + c * DCH, DCH)])


def _combine_kernel(g0_ref, g1_ref, w_ref, out_ref):
    w = w_ref[...]
    out_ref[...] = (w[:, 0:1] * g0_ref[0] + w[:, 1:2] * g1_ref[0])


def _combine_call(g2, w):
    bt = 512
    return pl.pallas_call(
        _combine_kernel,
        grid=(NUM_TOKENS // bt,),
        in_specs=[
            pl.BlockSpec((1, bt, D_MODEL), lambda t: (0, t, 0)),
            pl.BlockSpec((1, bt, D_MODEL), lambda t: (1, t, 0)),
            pl.BlockSpec((bt, 2), lambda t: (t, 0)),
        ],
        out_specs=pl.BlockSpec((bt, D_MODEL), lambda t: (t, 0)),
        out_shape=jax.ShapeDtypeStruct((NUM_TOKENS, D_MODEL), jnp.float32),
    )(g2, g2, w)


@jax.jit
def kernel(hidden_states, Wg, bg, W_gate_up, b_gate_up, W_down, b_down):
    Wf = W_gate_up.astype(jnp.bfloat16)
    bf2 = b_gate_up.reshape(NUM_EXPERTS, 1, 2 * D_FF)
    Wd = W_down.astype(jnp.bfloat16)
    bd2 = b_down.reshape(NUM_EXPERTS, 1, D_MODEL)
    bg2 = bg.reshape(1, NUM_EXPERTS)

    pos, w, be = _router_call(hidden_states, Wg, bg2)
    idx3 = pos.reshape(NWORK, TPW, 2).transpose(0, 2, 1)
    be_flat = be.reshape(NBLK)

    xs = _dispatch_kernel(hidden_states, idx3)
    rows = _gm_call(be_flat, xs, Wf, bf2, Wd, bd2)
    g2 = _gather_kernel(rows, idx3)
    return _combine_call(g2, w)
